# baseline (speedup 1.0000x reference)
import functools

import jax
import jax.numpy as jnp
from jax import lax
from jax.experimental import pallas as pl
from jax.experimental.pallas import tpu as pltpu
from jax.experimental.pallas import tpu_sc as plsc

D = 128
CHUNK = 128
NC = 2
NS = 16
NW = NC * NS
IDXB = 16
DEGW = 128


def _mesh():
    return plsc.VectorSubcoreMesh(core_axis_name="c", subcore_axis_name="s")


def _make_deg_kernel(n_pad, cpt):
    slab = n_pad // NS

    @functools.partial(
        pl.kernel,
        out_type=jax.ShapeDtypeStruct((NC, n_pad, DEGW), jnp.float32),
        mesh=_mesh(),
        scratch_types=[
            pltpu.VMEM((cpt, CHUNK), jnp.int32),
            pltpu.VMEM((CHUNK, DEGW), jnp.float32),
            pltpu.VMEM_SHARED((n_pad, DEGW), jnp.float32),
        ],
    )
    def deg_kernel(col_hbm, ones_hbm, zeros_hbm, out_hbm, idx_v, ones_v, deg_sh):
        cid = lax.axis_index("c")
        sid = lax.axis_index("s")
        gid = cid * NS + sid
        pltpu.sync_copy(zeros_hbm, deg_sh.at[pl.ds(sid * slab, slab)])
        pltpu.sync_copy(ones_hbm, ones_v)
        pltpu.sync_copy(col_hbm.at[pl.ds(gid * cpt, cpt)], idx_v)
        plsc.subcore_barrier()

        def body(j, carry):
            pltpu.sync_copy(ones_v, deg_sh.at[idx_v.at[j]], add=True)
            return carry

        lax.fori_loop(0, cpt, body, 0)
        plsc.subcore_barrier()
        pltpu.sync_copy(
            deg_sh.at[pl.ds(sid * slab, slab)],
            out_hbm.at[cid, pl.ds(sid * slab, slab)],
        )

    return deg_kernel


def _make_scatter_kernel(n_pad, cpt2):
    slab = n_pad // NS

    @functools.partial(
        pl.kernel,
        out_type=jax.ShapeDtypeStruct((n_pad, D), jnp.float32),
        mesh=_mesh(),
        scratch_types=[
            pltpu.VMEM((2, IDXB, CHUNK), jnp.int32),
            pltpu.VMEM((IDXB, CHUNK), jnp.int32),
            pltpu.VMEM((CHUNK, D), jnp.float32),
            pltpu.VMEM((CHUNK, D), jnp.float32),
            pltpu.SemaphoreType.DMA,
            pltpu.SemaphoreType.DMA,
            pltpu.VMEM_SHARED((n_pad, D), jnp.float32),
        ],
    )
    def scatter_kernel(y_hbm, row_hbm, col_hbm, zeros_hbm, out_hbm,
                       ridx_v, cidx_v, buf_a, buf_b, sem_a, sem_b, acc_sh):
        cid = lax.axis_index("c")
        sid = lax.axis_index("s")
        my_start = sid * cpt2

        @pl.when(cid == 0)
        def _():
            pltpu.sync_copy(zeros_hbm, acc_sh.at[pl.ds(sid * slab, slab)])
            pltpu.sync_copy(row_hbm.at[pl.ds(my_start, IDXB)], ridx_v.at[0])
            pltpu.sync_copy(col_hbm.at[pl.ds(my_start, IDXB)], cidx_v)

        plsc.subcore_barrier()

        bufs = (buf_a, buf_b)
        sems = (sem_a, sem_b)

        def ridx_at(j):
            return ridx_v.at[lax.rem(j // IDXB, 2), lax.rem(j, IDXB)]

        def gather(j, b):
            pltpu.async_copy(y_hbm.at[ridx_at(j)], bufs[b], sems[b])

        def gather_wait(j, b):
            pltpu.make_async_copy(y_hbm.at[ridx_at(j)], bufs[b], sems[b]).wait()

        @pl.when(cid == 0)
        def _():
            gather(0, 0)
            gather(1, 1)

            def body(j2, carry):
                j = 2 * j2
                blk = j // IDXB

                @pl.when(lax.rem(j, IDXB) == 0)
                def _():
                    @pl.when(j > 0)
                    def _():
                        pltpu.sync_copy(
                            col_hbm.at[pl.ds(my_start + blk * IDXB, IDXB)],
                            cidx_v)

                    @pl.when((blk + 1) * IDXB < cpt2)
                    def _():
                        pltpu.sync_copy(
                            row_hbm.at[pl.ds(my_start + (blk + 1) * IDXB, IDXB)],
                            ridx_v.at[lax.rem(blk + 1, 2)])

                for b in (0, 1):
                    jj = j + b
                    gather_wait(jj, b)
                    pltpu.sync_copy(
                        bufs[b], acc_sh.at[cidx_v.at[lax.rem(jj, IDXB)]],
                        add=True)

                    @pl.when(jj + 2 < cpt2)
                    def _():
                        gather(jj + 2, b)

                return carry

            lax.fori_loop(0, cpt2 // 2, body, 0)

        plsc.subcore_barrier()

        @pl.when(cid == 0)
        def _():
            pltpu.sync_copy(
                acc_sh.at[pl.ds(sid * slab, slab)],
                out_hbm.at[pl.ds(sid * slab, slab)],
            )

    return scatter_kernel


def _linear_body(x_ref, w_ref, degs_ref, y_ref, dis_ref):
    d16 = degs_ref[0] + degs_ref[1]
    deg = jnp.sum(d16, axis=1, keepdims=True) * (1.0 / DEGW) + 1.0
    dis = lax.rsqrt(deg)
    xw = jnp.dot(x_ref[...], w_ref[...], preferred_element_type=jnp.float32)
    y_ref[...] = xw * dis
    dis_ref[...] = dis


def _post_body(n, accs_ref, y_ref, dis_ref, b_ref, gamma_ref, beta_ref, out_ref):
    a = accs_ref[...] + y_ref[...]
    pre = dis_ref[...][:n] * a[:n] + b_ref[...]
    act = jnp.where(pre >= 0, pre, 0.01 * pre)
    mean = jnp.mean(act, axis=0, keepdims=True)
    var = jnp.mean((act - mean) ** 2, axis=0, keepdims=True)
    out_ref[...] = (act - mean) * lax.rsqrt(var + 1e-5) * gamma_ref[...] + beta_ref[...]


def kernel(x, edge_index, W, b, gamma, beta):
    n, d_in = x.shape
    d_out = W.shape[1]
    e = edge_index.shape[1]
    n_pad = ((n + 1) + 8 * NS - 1) // (8 * NS) * (8 * NS)
    cpt = (e + NW * CHUNK - 1) // (NW * CHUNK)
    cpt = (cpt + IDXB - 1) // IDXB * IDXB
    e_pad = NW * CHUNK * cpt

    ei = edge_index.astype(jnp.int32)
    pad = jnp.full((e_pad - e,), n, dtype=jnp.int32)
    row2d = jnp.concatenate([ei[0], pad]).reshape(-1, CHUNK)
    col2d = jnp.concatenate([ei[1], pad]).reshape(-1, CHUNK)
    x_pad = jnp.pad(x, ((0, n_pad - n), (0, 0)))

    slab = n_pad // NS
    ones16 = jnp.ones((CHUNK, DEGW), jnp.float32)
    zeros16 = jnp.zeros((slab, DEGW), jnp.float32)
    zerosd = jnp.zeros((slab, D), jnp.float32)

    degs = _make_deg_kernel(n_pad, cpt)(col2d, ones16, zeros16)

    y, dis = pl.pallas_call(
        _linear_body,
        out_shape=[
            jax.ShapeDtypeStruct((n_pad, d_out), jnp.float32),
            jax.ShapeDtypeStruct((n_pad, 1), jnp.float32),
        ],
    )(x_pad, W, degs)

    accs = _make_scatter_kernel(n_pad, 2 * cpt)(y, row2d, col2d, zerosd)

    out = pl.pallas_call(
        functools.partial(_post_body, n),
        out_shape=jax.ShapeDtypeStruct((n, d_out), jnp.float32),
    )(accs, y, dis, b.reshape(1, -1), gamma.reshape(1, -1), beta.reshape(1, -1))
    return out


# speedup vs baseline: 2.8144x; 2.8144x over previous
import functools

import jax
import jax.numpy as jnp
from jax import lax
from jax.experimental import pallas as pl
from jax.experimental.pallas import tpu as pltpu
from jax.experimental.pallas import tpu_sc as plsc

D = 128
CHUNK = 128
NC = 2
NS = 16
NW = NC * NS
IDXB = 16
DEGW = 128


def _mesh():
    return plsc.VectorSubcoreMesh(core_axis_name="c", subcore_axis_name="s")


def _make_deg_kernel(n_pad, cpt):
    slab = n_pad // NS

    @functools.partial(
        pl.kernel,
        out_type=jax.ShapeDtypeStruct((NC, n_pad, DEGW), jnp.float32),
        mesh=_mesh(),
        scratch_types=[
            pltpu.VMEM((cpt, CHUNK), jnp.int32),
            pltpu.VMEM((CHUNK, DEGW), jnp.float32),
            pltpu.VMEM_SHARED((n_pad, DEGW), jnp.float32),
        ],
    )
    def deg_kernel(col_hbm, ones_hbm, zeros_hbm, out_hbm, idx_v, ones_v, deg_sh):
        cid = lax.axis_index("c")
        sid = lax.axis_index("s")
        gid = cid * NS + sid
        pltpu.sync_copy(zeros_hbm, deg_sh.at[pl.ds(sid * slab, slab)])
        pltpu.sync_copy(ones_hbm, ones_v)
        pltpu.sync_copy(col_hbm.at[pl.ds(gid * cpt, cpt)], idx_v)
        plsc.subcore_barrier()

        def body(j, carry):
            pltpu.sync_copy(ones_v, deg_sh.at[idx_v.at[j]], add=True)
            return carry

        lax.fori_loop(0, cpt, body, 0)
        plsc.subcore_barrier()
        pltpu.sync_copy(
            deg_sh.at[pl.ds(sid * slab, slab)],
            out_hbm.at[cid, pl.ds(sid * slab, slab)],
        )

    return deg_kernel


def _make_scatter_kernel(n_pad, cpt2):
    slab = n_pad // NS

    @functools.partial(
        pl.kernel,
        out_type=jax.ShapeDtypeStruct((NC, n_pad, D), jnp.float32),
        mesh=_mesh(),
        scratch_types=[
            pltpu.VMEM((2, IDXB, CHUNK), jnp.int32),
            pltpu.VMEM((IDXB, CHUNK), jnp.int32),
            pltpu.VMEM((CHUNK, D), jnp.float32),
            pltpu.VMEM((CHUNK, D), jnp.float32),
            pltpu.SemaphoreType.DMA,
            pltpu.SemaphoreType.DMA,
            pltpu.VMEM_SHARED((n_pad, D), jnp.float32),
        ],
    )
    def scatter_kernel(y_hbm, row_hbm, col_hbm, zeros_hbm, out_hbm,
                       ridx_v, cidx_v, buf_a, buf_b, sem_a, sem_b, acc_sh):
        cid = lax.axis_index("c")
        sid = lax.axis_index("s")
        gid = cid * NS + sid
        my_start = gid * cpt2
        pltpu.sync_copy(zeros_hbm, acc_sh.at[pl.ds(sid * slab, slab)])
        pltpu.sync_copy(row_hbm.at[pl.ds(my_start, IDXB)], ridx_v.at[0])
        pltpu.sync_copy(col_hbm.at[pl.ds(my_start, IDXB)], cidx_v)
        plsc.subcore_barrier()

        bufs = (buf_a, buf_b)
        sems = (sem_a, sem_b)

        def ridx_at(j):
            return ridx_v.at[lax.rem(j // IDXB, 2), lax.rem(j, IDXB)]

        def gather(j, b):
            pltpu.async_copy(y_hbm.at[ridx_at(j)], bufs[b], sems[b])

        def gather_wait(j, b):
            pltpu.make_async_copy(y_hbm.at[ridx_at(j)], bufs[b], sems[b]).wait()

        gather(0, 0)
        gather(1, 1)

        def body(j2, carry):
            j = 2 * j2
            blk = j // IDXB

            @pl.when(lax.rem(j, IDXB) == 0)
            def _():
                @pl.when(j > 0)
                def _():
                    pltpu.sync_copy(
                        col_hbm.at[pl.ds(my_start + blk * IDXB, IDXB)], cidx_v)

                @pl.when((blk + 1) * IDXB < cpt2)
                def _():
                    pltpu.sync_copy(
                        row_hbm.at[pl.ds(my_start + (blk + 1) * IDXB, IDXB)],
                        ridx_v.at[lax.rem(blk + 1, 2)])

            for b in (0, 1):
                jj = j + b
                gather_wait(jj, b)
                pltpu.sync_copy(
                    bufs[b], acc_sh.at[cidx_v.at[lax.rem(jj, IDXB)]], add=True)

                @pl.when(jj + 2 < cpt2)
                def _():
                    gather(jj + 2, b)

            return carry

        lax.fori_loop(0, cpt2 // 2, body, 0)
        plsc.subcore_barrier()
        pltpu.sync_copy(
            acc_sh.at[pl.ds(sid * slab, slab)],
            out_hbm.at[cid, pl.ds(sid * slab, slab)],
        )

    return scatter_kernel


def _linear_body(x_ref, w_ref, degs_ref, y_ref, dis_ref):
    d16 = degs_ref[0] + degs_ref[1]
    deg = jnp.sum(d16, axis=1, keepdims=True) * (1.0 / DEGW) + 1.0
    dis = lax.rsqrt(deg)
    xw = jnp.dot(x_ref[...], w_ref[...], preferred_element_type=jnp.float32)
    y_ref[...] = xw * dis
    dis_ref[...] = dis


def _post_body(n, accs_ref, y_ref, dis_ref, b_ref, gamma_ref, beta_ref, out_ref):
    a = accs_ref[0] + accs_ref[1] + y_ref[...]
    pre = dis_ref[...][:n] * a[:n] + b_ref[...]
    act = jnp.where(pre >= 0, pre, 0.01 * pre)
    mean = jnp.mean(act, axis=0, keepdims=True)
    var = jnp.mean((act - mean) ** 2, axis=0, keepdims=True)
    out_ref[...] = (act - mean) * lax.rsqrt(var + 1e-5) * gamma_ref[...] + beta_ref[...]


def kernel(x, edge_index, W, b, gamma, beta):
    n, d_in = x.shape
    d_out = W.shape[1]
    e = edge_index.shape[1]
    n_pad = ((n + 1) + 8 * NS - 1) // (8 * NS) * (8 * NS)
    cpt = (e + NW * CHUNK - 1) // (NW * CHUNK)
    cpt = (cpt + IDXB - 1) // IDXB * IDXB
    e_pad = NW * CHUNK * cpt

    ei = edge_index.astype(jnp.int32)
    padn = e_pad - e
    ar = jnp.arange(padn, dtype=jnp.int32)
    row_pad = ar % n
    col_pad = n + 1 + ar % (n_pad - n - 1)
    row2d = jnp.concatenate([ei[0], row_pad]).reshape(-1, CHUNK)
    col2d = jnp.concatenate([ei[1], col_pad]).reshape(-1, CHUNK)
    x_pad = jnp.pad(x, ((0, n_pad - n), (0, 0)))

    slab = n_pad // NS
    ones16 = jnp.ones((CHUNK, DEGW), jnp.float32)
    zeros16 = jnp.zeros((slab, DEGW), jnp.float32)
    zerosd = jnp.zeros((slab, D), jnp.float32)

    degs = _make_deg_kernel(n_pad, cpt)(col2d, ones16, zeros16)

    y, dis = pl.pallas_call(
        _linear_body,
        out_shape=[
            jax.ShapeDtypeStruct((n_pad, d_out), jnp.float32),
            jax.ShapeDtypeStruct((n_pad, 1), jnp.float32),
        ],
    )(x_pad, W, degs)

    accs = _make_scatter_kernel(n_pad, cpt)(y, row2d, col2d, zerosd)

    out = pl.pallas_call(
        functools.partial(_post_body, n),
        out_shape=jax.ShapeDtypeStruct((n, d_out), jnp.float32),
    )(accs, y, dis, b.reshape(1, -1), gamma.reshape(1, -1), beta.reshape(1, -1))
    return out


# speedup vs baseline: 3.6924x; 1.3120x over previous
import functools

import jax
import jax.numpy as jnp
from jax import lax
from jax.experimental import pallas as pl
from jax.experimental.pallas import tpu as pltpu
from jax.experimental.pallas import tpu_sc as plsc

D = 128
CHUNK = 128
NC = 2
NS = 16
NW = NC * NS
IDXB = 16
DEGW = 128


def _mesh():
    return plsc.VectorSubcoreMesh(core_axis_name="c", subcore_axis_name="s")


def _make_deg_kernel(n_pad, cpt):
    slab = n_pad // NS

    @functools.partial(
        pl.kernel,
        out_type=jax.ShapeDtypeStruct((NC * n_pad,), jnp.float32),
        mesh=_mesh(),
        scratch_types=[
            pltpu.VMEM((cpt, CHUNK), jnp.int32),
            pltpu.VMEM((CHUNK,), jnp.float32),
            pltpu.VMEM_SHARED((n_pad,), jnp.float32),
        ],
    )
    def deg_kernel(col_hbm, ones_hbm, zeros_hbm, out_hbm, idx_v, ones_v, deg_sh):
        cid = lax.axis_index("c")
        sid = lax.axis_index("s")
        gid = cid * NS + sid
        pltpu.sync_copy(zeros_hbm, deg_sh.at[pl.ds(sid * slab, slab)])
        pltpu.sync_copy(ones_hbm, ones_v)
        pltpu.sync_copy(col_hbm.at[pl.ds(gid * cpt, cpt)], idx_v)
        plsc.subcore_barrier()

        def body(j, carry):
            pltpu.sync_copy(ones_v, deg_sh.at[idx_v.at[j]], add=True)
            return carry

        lax.fori_loop(0, cpt, body, 0)
        plsc.subcore_barrier()
        pltpu.sync_copy(
            deg_sh.at[pl.ds(sid * slab, slab)],
            out_hbm.at[pl.ds(cid * n_pad + sid * slab, slab)],
        )

    return deg_kernel


def _make_scatter_kernel(n_pad, cpt2):
    slab = n_pad // NS

    @functools.partial(
        pl.kernel,
        out_type=jax.ShapeDtypeStruct((NC, n_pad, D), jnp.float32),
        mesh=_mesh(),
        scratch_types=[
            pltpu.VMEM((2, IDXB, CHUNK), jnp.int32),
            pltpu.VMEM((IDXB, CHUNK), jnp.int32),
            pltpu.VMEM((CHUNK, D), jnp.float32),
            pltpu.VMEM((CHUNK, D), jnp.float32),
            pltpu.SemaphoreType.DMA,
            pltpu.SemaphoreType.DMA,
            pltpu.VMEM_SHARED((n_pad, D), jnp.float32),
        ],
    )
    def scatter_kernel(y_hbm, row_hbm, col_hbm, zeros_hbm, out_hbm,
                       ridx_v, cidx_v, buf_a, buf_b, sem_a, sem_b, acc_sh):
        cid = lax.axis_index("c")
        sid = lax.axis_index("s")
        gid = cid * NS + sid
        my_start = gid * cpt2
        pltpu.sync_copy(zeros_hbm, acc_sh.at[pl.ds(sid * slab, slab)])
        pltpu.sync_copy(row_hbm.at[pl.ds(my_start, IDXB)], ridx_v.at[0])
        pltpu.sync_copy(col_hbm.at[pl.ds(my_start, IDXB)], cidx_v)
        plsc.subcore_barrier()

        bufs = (buf_a, buf_b)
        sems = (sem_a, sem_b)

        def ridx_at(j):
            return ridx_v.at[lax.rem(j // IDXB, 2), lax.rem(j, IDXB)]

        def gather(j, b):
            pltpu.async_copy(y_hbm.at[ridx_at(j)], bufs[b], sems[b])

        def gather_wait(j, b):
            pltpu.make_async_copy(y_hbm.at[ridx_at(j)], bufs[b], sems[b]).wait()

        gather(0, 0)
        gather(1, 1)

        def body(j2, carry):
            j = 2 * j2
            blk = j // IDXB

            @pl.when(lax.rem(j, IDXB) == 0)
            def _():
                @pl.when(j > 0)
                def _():
                    pltpu.sync_copy(
                        col_hbm.at[pl.ds(my_start + blk * IDXB, IDXB)], cidx_v)

                @pl.when((blk + 1) * IDXB < cpt2)
                def _():
                    pltpu.sync_copy(
                        row_hbm.at[pl.ds(my_start + (blk + 1) * IDXB, IDXB)],
                        ridx_v.at[lax.rem(blk + 1, 2)])

            for b in (0, 1):
                jj = j + b
                gather_wait(jj, b)
                pltpu.sync_copy(
                    bufs[b], acc_sh.at[cidx_v.at[lax.rem(jj, IDXB)]], add=True)

                @pl.when(jj + 2 < cpt2)
                def _():
                    gather(jj + 2, b)

            return carry

        lax.fori_loop(0, cpt2 // 2, body, 0)
        plsc.subcore_barrier()
        pltpu.sync_copy(
            acc_sh.at[pl.ds(sid * slab, slab)],
            out_hbm.at[cid, pl.ds(sid * slab, slab)],
        )

    return scatter_kernel


def _linear_body(n_pad, x_ref, w_ref, deg_ref, y_ref, dis_ref):
    dis = lax.rsqrt(deg_ref[...] + 1.0)
    xw = jnp.dot(x_ref[...], w_ref[...], preferred_element_type=jnp.float32)
    n = x_ref.shape[0]
    y_ref[...] = jnp.pad(xw, ((0, n_pad - n), (0, 0))) * dis
    dis_ref[...] = dis


def _post_body(n, accs_ref, y_ref, dis_ref, b_ref, gamma_ref, beta_ref, out_ref):
    a = accs_ref[0] + accs_ref[1] + y_ref[...]
    pre = dis_ref[...][:n] * a[:n] + b_ref[...]
    act = jnp.where(pre >= 0, pre, 0.01 * pre)
    mean = jnp.mean(act, axis=0, keepdims=True)
    var = jnp.mean((act - mean) ** 2, axis=0, keepdims=True)
    out_ref[...] = (act - mean) * lax.rsqrt(var + 1e-5) * gamma_ref[...] + beta_ref[...]


def kernel(x, edge_index, W, b, gamma, beta):
    n, d_in = x.shape
    d_out = W.shape[1]
    e = edge_index.shape[1]
    n_pad = ((n + 1) + 128 * NS - 1) // (128 * NS) * (128 * NS)
    cpt = (e + NW * CHUNK - 1) // (NW * CHUNK)
    cpt = (cpt + IDXB - 1) // IDXB * IDXB
    e_pad = NW * CHUNK * cpt

    ei = edge_index.astype(jnp.int32)
    padn = e_pad - e
    ar = jnp.arange(padn, dtype=jnp.int32)
    row_pad = ar % n
    col_pad = n + 1 + ar % (n_pad - n - 1)
    row2d = jnp.concatenate([ei[0], row_pad]).reshape(-1, CHUNK)
    col2d = jnp.concatenate([ei[1], col_pad]).reshape(-1, CHUNK)

    slab = n_pad // NS
    ones1 = jnp.ones((CHUNK,), jnp.float32)
    zeros1 = jnp.zeros((slab,), jnp.float32)
    zerosd = jnp.zeros((slab, D), jnp.float32)

    degs = _make_deg_kernel(n_pad, cpt)(col2d, ones1, zeros1)
    deg_col = degs.reshape(NC, n_pad).sum(0).reshape(n_pad, 1)

    y, dis = pl.pallas_call(
        functools.partial(_linear_body, n_pad),
        out_shape=[
            jax.ShapeDtypeStruct((n_pad, d_out), jnp.float32),
            jax.ShapeDtypeStruct((n_pad, 1), jnp.float32),
        ],
    )(x, W, deg_col)

    accs = _make_scatter_kernel(n_pad, cpt)(y, row2d, col2d, zerosd)

    out = pl.pallas_call(
        functools.partial(_post_body, n),
        out_shape=jax.ShapeDtypeStruct((n, d_out), jnp.float32),
    )(accs, y, dis, b.reshape(1, -1), gamma.reshape(1, -1), beta.reshape(1, -1))
    return out


# speedup vs baseline: 3.9067x; 1.0580x over previous
import functools

import jax
import jax.numpy as jnp
from jax import lax
from jax.experimental import pallas as pl
from jax.experimental.pallas import tpu as pltpu
from jax.experimental.pallas import tpu_sc as plsc

D = 128
CHUNK = 128
NC = 2
NS = 16
NW = NC * NS
IDXB = 16
DEGW = 128


def _mesh():
    return plsc.VectorSubcoreMesh(core_axis_name="c", subcore_axis_name="s")


def _make_deg_kernel(n_pad, cpt, main_chunks):
    slab = n_pad // NS
    btile = main_chunks // cpt
    m1 = main_chunks - btile * cpt

    @functools.partial(
        pl.kernel,
        out_type=jax.ShapeDtypeStruct((NC * n_pad,), jnp.float32),
        mesh=_mesh(),
        scratch_types=[
            pltpu.VMEM((cpt, CHUNK), jnp.int32),
            pltpu.VMEM((CHUNK,), jnp.float32),
            pltpu.VMEM_SHARED((n_pad,), jnp.float32),
        ],
    )
    def deg_kernel(ei_hbm, tail_hbm, ones_hbm, zeros_hbm, out_hbm,
                   idx_v, ones_v, deg_sh):
        cid = lax.axis_index("c")
        sid = lax.axis_index("s")
        gid = cid * NS + sid
        pltpu.sync_copy(zeros_hbm, deg_sh.at[pl.ds(sid * slab, slab)])
        pltpu.sync_copy(ones_hbm, ones_v)

        @pl.when(gid < btile)
        def _():
            pltpu.sync_copy(ei_hbm.at[1, pl.ds(gid * cpt, cpt)], idx_v)

        @pl.when(gid == btile)
        def _():
            if m1 > 0:
                pltpu.sync_copy(ei_hbm.at[1, pl.ds(btile * cpt, m1)],
                                idx_v.at[pl.ds(0, m1)])
            pltpu.sync_copy(tail_hbm.at[1, pl.ds(0, cpt - m1)],
                            idx_v.at[pl.ds(m1, cpt - m1)])

        @pl.when(gid > btile)
        def _():
            pltpu.sync_copy(
                tail_hbm.at[1, pl.ds(gid * cpt - main_chunks, cpt)], idx_v)

        plsc.subcore_barrier()

        def body(j, carry):
            pltpu.sync_copy(ones_v, deg_sh.at[idx_v.at[j]], add=True)
            return carry

        lax.fori_loop(0, cpt, body, 0)
        plsc.subcore_barrier()
        pltpu.sync_copy(
            deg_sh.at[pl.ds(sid * slab, slab)],
            out_hbm.at[pl.ds(cid * n_pad + sid * slab, slab)],
        )

    return deg_kernel


def _make_scatter_kernel(n_pad, cpt2, main_chunks):
    slab = n_pad // NS

    @functools.partial(
        pl.kernel,
        out_type=jax.ShapeDtypeStruct((NC, n_pad, D), jnp.float32),
        mesh=_mesh(),
        scratch_types=[
            pltpu.VMEM((2, IDXB, CHUNK), jnp.int32),
            pltpu.VMEM((IDXB, CHUNK), jnp.int32),
            pltpu.VMEM((CHUNK, D), jnp.float32),
            pltpu.VMEM((CHUNK, D), jnp.float32),
            pltpu.SemaphoreType.DMA,
            pltpu.SemaphoreType.DMA,
            pltpu.VMEM_SHARED((n_pad, D), jnp.float32),
        ],
    )
    def scatter_kernel(y_hbm, ei_hbm, tail_hbm, zeros_hbm, out_hbm,
                       ridx_v, cidx_v, buf_a, buf_b, sem_a, sem_b, acc_sh):
        cid = lax.axis_index("c")
        sid = lax.axis_index("s")
        gid = cid * NS + sid
        my_start = gid * cpt2

        def load_idx_block(rc, g0, dst):
            @pl.when(g0 < main_chunks)
            def _():
                pltpu.sync_copy(ei_hbm.at[rc, pl.ds(g0, IDXB)], dst)

            @pl.when(g0 >= main_chunks)
            def _():
                pltpu.sync_copy(tail_hbm.at[rc, pl.ds(g0 - main_chunks, IDXB)],
                                dst)

        pltpu.sync_copy(zeros_hbm, acc_sh.at[pl.ds(sid * slab, slab)])
        load_idx_block(0, my_start, ridx_v.at[0])
        load_idx_block(1, my_start, cidx_v)
        plsc.subcore_barrier()

        bufs = (buf_a, buf_b)
        sems = (sem_a, sem_b)

        def ridx_at(j):
            return ridx_v.at[lax.rem(j // IDXB, 2), lax.rem(j, IDXB)]

        def gather(j, b):
            pltpu.async_copy(y_hbm.at[ridx_at(j)], bufs[b], sems[b])

        def gather_wait(j, b):
            pltpu.make_async_copy(y_hbm.at[ridx_at(j)], bufs[b], sems[b]).wait()

        gather(0, 0)
        gather(1, 1)

        def body(j2, carry):
            j = 2 * j2
            blk = j // IDXB

            @pl.when(lax.rem(j, IDXB) == 0)
            def _():
                @pl.when(j > 0)
                def _():
                    load_idx_block(1, my_start + blk * IDXB, cidx_v)

                @pl.when((blk + 1) * IDXB < cpt2)
                def _():
                    load_idx_block(0, my_start + (blk + 1) * IDXB,
                                   ridx_v.at[lax.rem(blk + 1, 2)])

            for b in (0, 1):
                jj = j + b
                gather_wait(jj, b)
                pltpu.sync_copy(
                    bufs[b], acc_sh.at[cidx_v.at[lax.rem(jj, IDXB)]], add=True)

                @pl.when(jj + 2 < cpt2)
                def _():
                    gather(jj + 2, b)

            return carry

        lax.fori_loop(0, cpt2 // 2, body, 0)
        plsc.subcore_barrier()
        pltpu.sync_copy(
            acc_sh.at[pl.ds(sid * slab, slab)],
            out_hbm.at[cid, pl.ds(sid * slab, slab)],
        )

    return scatter_kernel


def _linear_body(n_pad, x_ref, w_ref, deg_ref, y_ref, dis_ref):
    dis = lax.rsqrt(deg_ref[...] + 1.0)
    xw = jnp.dot(x_ref[...], w_ref[...], preferred_element_type=jnp.float32)
    n = x_ref.shape[0]
    y_ref[...] = jnp.pad(xw, ((0, n_pad - n), (0, 0))) * dis
    dis_ref[...] = dis


def _post_body(n, accs_ref, y_ref, dis_ref, b_ref, gamma_ref, beta_ref, out_ref):
    a = accs_ref[0] + accs_ref[1] + y_ref[...]
    pre = dis_ref[...][:n] * a[:n] + b_ref[...]
    act = jnp.where(pre >= 0, pre, 0.01 * pre)
    mean = jnp.mean(act, axis=0, keepdims=True)
    var = jnp.mean((act - mean) ** 2, axis=0, keepdims=True)
    out_ref[...] = (act - mean) * lax.rsqrt(var + 1e-5) * gamma_ref[...] + beta_ref[...]


def kernel(x, edge_index, W, b, gamma, beta):
    n, d_in = x.shape
    d_out = W.shape[1]
    e = edge_index.shape[1]
    n_pad = ((n + 1) + 128 * NS - 1) // (128 * NS) * (128 * NS)
    cpt = (e + NW * CHUNK - 1) // (NW * CHUNK)
    cpt = (cpt + IDXB - 1) // IDXB * IDXB
    e_pad = NW * CHUNK * cpt

    ei = edge_index.astype(jnp.int32)
    main_chunks = e // CHUNK // IDXB * IDXB
    padn = e_pad - e
    ar = jnp.arange(padn, dtype=jnp.int32)
    row_pad = ar % n
    col_pad = n + 1 + ar % (n_pad - n - 1)
    ei3 = ei.reshape(2, -1, CHUNK) if e % CHUNK == 0 else (
        ei[:, :e // CHUNK * CHUNK].reshape(2, -1, CHUNK))
    tail3 = jnp.concatenate(
        [ei[:, main_chunks * CHUNK:], jnp.stack([row_pad, col_pad])],
        axis=1).reshape(2, -1, CHUNK)

    slab = n_pad // NS
    ones1 = jnp.ones((CHUNK,), jnp.float32)
    zeros1 = jnp.zeros((slab,), jnp.float32)
    zerosd = jnp.zeros((slab, D), jnp.float32)

    degs = _make_deg_kernel(n_pad, cpt, main_chunks)(ei3, tail3, ones1, zeros1)
    deg_col = degs.reshape(NC, n_pad).sum(0).reshape(n_pad, 1)

    y, dis = pl.pallas_call(
        functools.partial(_linear_body, n_pad),
        out_shape=[
            jax.ShapeDtypeStruct((n_pad, d_out), jnp.float32),
            jax.ShapeDtypeStruct((n_pad, 1), jnp.float32),
        ],
    )(x, W, deg_col)

    accs = _make_scatter_kernel(n_pad, cpt, main_chunks)(y, ei3, tail3, zerosd)

    out = pl.pallas_call(
        functools.partial(_post_body, n),
        out_shape=jax.ShapeDtypeStruct((n, d_out), jnp.float32),
    )(accs, y, dis, b.reshape(1, -1), gamma.reshape(1, -1), beta.reshape(1, -1))
    return out


# speedup vs baseline: 3.9085x; 1.0004x over previous
import functools

import jax
import jax.numpy as jnp
from jax import lax
from jax.experimental import pallas as pl
from jax.experimental.pallas import tpu as pltpu
from jax.experimental.pallas import tpu_sc as plsc

D = 128
CHUNK = 128
NC = 2
NS = 16
NW = NC * NS
IDXB = 16


def _mesh():
    return plsc.VectorSubcoreMesh(core_axis_name="c", subcore_axis_name="s")


def _make_deg_kernel(n_pad, cpt, main_chunks):
    slab = n_pad // NS
    btile = main_chunks // cpt
    m1 = main_chunks - btile * cpt

    @functools.partial(
        pl.kernel,
        out_type=jax.ShapeDtypeStruct((NC * n_pad,), jnp.float32),
        mesh=_mesh(),
        scratch_types=[
            pltpu.VMEM((cpt, CHUNK), jnp.int32),
            pltpu.VMEM((CHUNK,), jnp.float32),
            pltpu.VMEM_SHARED((n_pad,), jnp.float32),
        ],
    )
    def deg_kernel(ei_hbm, tail_hbm, ones_hbm, zeros_hbm, out_hbm,
                   idx_v, ones_v, deg_sh):
        cid = lax.axis_index("c")
        sid = lax.axis_index("s")
        gid = cid * NS + sid
        pltpu.sync_copy(zeros_hbm, deg_sh.at[pl.ds(sid * slab, slab)])
        pltpu.sync_copy(ones_hbm, ones_v)

        @pl.when(gid < btile)
        def _():
            pltpu.sync_copy(ei_hbm.at[1, pl.ds(gid * cpt, cpt)], idx_v)

        @pl.when(gid == btile)
        def _():
            if m1 > 0:
                pltpu.sync_copy(ei_hbm.at[1, pl.ds(btile * cpt, m1)],
                                idx_v.at[pl.ds(0, m1)])
            pltpu.sync_copy(tail_hbm.at[1, pl.ds(0, cpt - m1)],
                            idx_v.at[pl.ds(m1, cpt - m1)])

        @pl.when(gid > btile)
        def _():
            pltpu.sync_copy(
                tail_hbm.at[1, pl.ds(gid * cpt - main_chunks, cpt)], idx_v)

        plsc.subcore_barrier()

        def body(j, carry):
            pltpu.sync_copy(ones_v, deg_sh.at[idx_v.at[j]], add=True)
            return carry

        lax.fori_loop(0, cpt, body, 0)
        plsc.subcore_barrier()
        pltpu.sync_copy(
            deg_sh.at[pl.ds(sid * slab, slab)],
            out_hbm.at[pl.ds(cid * n_pad + sid * slab, slab)],
        )

    return deg_kernel


def _make_scatter_kernel(n_pad, cpt2, main_chunks):
    slab = n_pad // NS

    @functools.partial(
        pl.kernel,
        out_type=jax.ShapeDtypeStruct((NC, n_pad, D), jnp.float32),
        mesh=_mesh(),
        scratch_types=[
            pltpu.VMEM((2, IDXB, CHUNK), jnp.int32),
            pltpu.VMEM((IDXB, CHUNK), jnp.int32),
            pltpu.VMEM((CHUNK, D), jnp.float32),
            pltpu.VMEM((CHUNK, D), jnp.float32),
            pltpu.SemaphoreType.DMA,
            pltpu.SemaphoreType.DMA,
            pltpu.VMEM_SHARED((n_pad, D), jnp.float32),
        ],
    )
    def scatter_kernel(y_hbm, ei_hbm, tail_hbm, zeros_hbm, out_hbm,
                       ridx_v, cidx_v, buf_a, buf_b, sem_a, sem_b, acc_sh):
        cid = lax.axis_index("c")
        sid = lax.axis_index("s")
        gid = cid * NS + sid
        my_start = gid * cpt2

        def load_idx_block(rc, g0, dst):
            @pl.when(g0 < main_chunks)
            def _():
                pltpu.sync_copy(ei_hbm.at[rc, pl.ds(g0, IDXB)], dst)

            @pl.when(g0 >= main_chunks)
            def _():
                pltpu.sync_copy(tail_hbm.at[rc, pl.ds(g0 - main_chunks, IDXB)],
                                dst)

        pltpu.sync_copy(zeros_hbm, acc_sh.at[pl.ds(sid * slab, slab)])
        load_idx_block(0, my_start, ridx_v.at[0])
        load_idx_block(1, my_start, cidx_v)
        plsc.subcore_barrier()

        bufs = (buf_a, buf_b)
        sems = (sem_a, sem_b)

        def ridx_at(j):
            return ridx_v.at[lax.rem(j // IDXB, 2), lax.rem(j, IDXB)]

        def gather(j, b):
            pltpu.async_copy(y_hbm.at[ridx_at(j)], bufs[b], sems[b])

        def gather_wait(j, b):
            pltpu.make_async_copy(y_hbm.at[ridx_at(j)], bufs[b], sems[b]).wait()

        gather(0, 0)
        gather(1, 1)

        def body(j2, carry):
            j = 2 * j2
            blk = j // IDXB

            @pl.when(lax.rem(j, IDXB) == 0)
            def _():
                @pl.when(j > 0)
                def _():
                    load_idx_block(1, my_start + blk * IDXB, cidx_v)

                @pl.when((blk + 1) * IDXB < cpt2)
                def _():
                    load_idx_block(0, my_start + (blk + 1) * IDXB,
                                   ridx_v.at[lax.rem(blk + 1, 2)])

            for b in (0, 1):
                jj = j + b
                gather_wait(jj, b)
                pltpu.sync_copy(
                    bufs[b], acc_sh.at[cidx_v.at[lax.rem(jj, IDXB)]], add=True)

                @pl.when(jj + 2 < cpt2)
                def _():
                    gather(jj + 2, b)

            return carry

        lax.fori_loop(0, cpt2 // 2, body, 0)
        plsc.subcore_barrier()
        pltpu.sync_copy(
            acc_sh.at[pl.ds(sid * slab, slab)],
            out_hbm.at[cid, pl.ds(sid * slab, slab)],
        )

    return scatter_kernel


def _linear_body(n_pad, x_ref, w_ref, deg_ref, y_ref, dis_ref):
    dis = lax.rsqrt(deg_ref[...] + 1.0)
    xw = jnp.dot(x_ref[...], w_ref[...], preferred_element_type=jnp.float32)
    n = x_ref.shape[0]
    y_ref[...] = jnp.pad(xw, ((0, n_pad - n), (0, 0))) * dis
    dis_ref[...] = dis


def _post_body(n, accs_ref, y_ref, dis_ref, b_ref, gamma_ref, beta_ref, out_ref):
    a = accs_ref[0] + accs_ref[1] + y_ref[...]
    pre = dis_ref[...][:n] * a[:n] + b_ref[...]
    act = jnp.where(pre >= 0, pre, 0.01 * pre)
    mean = jnp.mean(act, axis=0, keepdims=True)
    var = jnp.mean((act - mean) ** 2, axis=0, keepdims=True)
    out_ref[...] = (act - mean) * lax.rsqrt(var + 1e-5) * gamma_ref[...] + beta_ref[...]


def kernel(x, edge_index, W, b, gamma, beta):
    n = x.shape[0]
    d_out = W.shape[1]
    e = edge_index.shape[1]
    n_pad = ((n + 1) + 128 * NS - 1) // (128 * NS) * (128 * NS)
    cpt = (e + NW * CHUNK - 1) // (NW * CHUNK)
    cpt = (cpt + IDXB - 1) // IDXB * IDXB
    e_pad = NW * CHUNK * cpt

    ei = edge_index.astype(jnp.int32)
    main_chunks = e // CHUNK // IDXB * IDXB
    padn = e_pad - e
    ar = jnp.arange(padn, dtype=jnp.int32)
    row_pad = ar % n
    col_pad = n + 1 + ar % (n_pad - n - 1)
    ei3 = ei.reshape(2, -1, CHUNK) if e % CHUNK == 0 else (
        ei[:, :e // CHUNK * CHUNK].reshape(2, -1, CHUNK))
    tail3 = jnp.concatenate(
        [ei[:, main_chunks * CHUNK:], jnp.stack([row_pad, col_pad])],
        axis=1).reshape(2, -1, CHUNK)

    slab = n_pad // NS
    ones1 = jnp.ones((CHUNK,), jnp.float32)
    zeros1 = jnp.zeros((slab,), jnp.float32)
    zerosd = jnp.zeros((slab, D), jnp.float32)

    degs = _make_deg_kernel(n_pad, cpt, main_chunks)(ei3, tail3, ones1, zeros1)
    deg_col = degs.reshape(NC, n_pad).sum(0).reshape(n_pad, 1)

    y, dis = pl.pallas_call(
        functools.partial(_linear_body, n_pad),
        out_shape=[
            jax.ShapeDtypeStruct((n_pad, d_out), jnp.float32),
            jax.ShapeDtypeStruct((n_pad, 1), jnp.float32),
        ],
    )(x, W, deg_col)

    accs = _make_scatter_kernel(n_pad, cpt, main_chunks)(y, ei3, tail3, zerosd)

    out = pl.pallas_call(
        functools.partial(_post_body, n),
        out_shape=jax.ShapeDtypeStruct((n, d_out), jnp.float32),
    )(accs, y, dis, b.reshape(1, -1), gamma.reshape(1, -1), beta.reshape(1, -1))
    return out
